# baseline (speedup 1.0000x reference)
import functools

import jax
import jax.numpy as jnp
from jax import lax
from jax.experimental import pallas as pl
from jax.experimental.pallas import tpu as pltpu
from jax.experimental.pallas import tpu_sc as plsc

_SC_FRAC = 0.5
_PN = 32
_NW = 32


def _pick_block(n, cap=2048):
    for b in range(cap - cap % 8, 0, -8):
        if n % b == 0:
            return b
    for b in range(min(n, cap), 0, -1):
        if n % b == 0:
            return b
    return n


def _pass1_kernel(h_ref, w_ref, bl_ref, a_ref, wexp_ref,
                  acc_ref, comp_ref, bd_ref):
    i = pl.program_id(0)
    c, bn, f = h_ref.shape

    @pl.when(i == 0)
    def _():
        acc_ref[...] = jnp.zeros_like(acc_ref)
        comp_ref[...] = jnp.zeros_like(comp_ref)
        row = lax.broadcasted_iota(jnp.int32, (c, c * bn), 0)
        col = lax.broadcasted_iota(jnp.int32, (c, c * bn), 1)
        bd_ref[...] = (col // bn == row).astype(jnp.bfloat16)

    x = h_ref[...].reshape(c * bn, f)
    s = jnp.tanh(
        jnp.dot(x, w_ref[...], preferred_element_type=jnp.float32) + bl_ref[...]
    )
    colsum = jnp.dot(bd_ref[...], s.astype(jnp.bfloat16),
                     preferred_element_type=jnp.float32)
    a_r = a_ref[...].astype(jnp.bfloat16).astype(jnp.float32)
    t = jnp.sum(colsum * a_r, axis=1)
    tb = jnp.broadcast_to(t[:, None], acc_ref.shape)
    y = tb - comp_ref[...]
    tot = acc_ref[...] + y
    comp_ref[...] = (tot - acc_ref[...]) - y
    acc_ref[...] = tot

    @pl.when(i == pl.num_programs(0) - 1)
    def _():
        b = acc_ref[...] * (1.0 / c)
        m = jnp.max(b, axis=0, keepdims=True)
        e = jnp.exp(b - m)
        wexp_ref[...] = e / jnp.sum(e, axis=0, keepdims=True)


def _pass2_tc_kernel(wexp_ref, h_ref, out_ref):
    c = h_ref.shape[0]
    w = wexp_ref[...]
    hb = h_ref[...]
    acc = hb[0] * w[0]
    for i in range(1, c):
        acc = acc + hb[i] * w[i]
    out_ref[...] = acc


def _make_sc_pass2(c, f, node0, npw, pieces, pn):
    mesh = plsc.VectorSubcoreMesh(core_axis_name="c", subcore_axis_name="s")
    nc = 2
    lanes = f // 16

    @functools.partial(
        pl.kernel,
        out_type=jax.ShapeDtypeStruct((npw * _NW, f), jnp.float32),
        mesh=mesh,
        scratch_types=[
            pltpu.VMEM((c, f), jnp.float32),
            [pltpu.VMEM((c, pn, f), jnp.float32) for _ in range(2)],
            pltpu.VMEM((pn, f), jnp.float32),
            [pltpu.SemaphoreType.DMA for _ in range(2)],
        ],
    )
    def sc_pass2(h_ref, wexp_ref, out_ref, wbuf, hbufs, accbuf, sems):
        wid = lax.axis_index("s") * nc + lax.axis_index("c")
        base = node0 + wid * npw
        obase = wid * npw
        pltpu.sync_copy(wexp_ref, wbuf)
        wv = [wbuf[i, 0:16] for i in range(c)]

        def copy(slot, p):
            src = base + p * pn
            hb, sem = hbufs[slot], sems[slot]

            class _Batch:
                def start(self):
                    for i in range(c):
                        pltpu.make_async_copy(
                            h_ref.at[i, pl.ds(src, pn), :], hb.at[i], sem
                        ).start()

                def wait(self):
                    for i in range(c):
                        pltpu.make_async_copy(
                            h_ref.at[i, pl.ds(src, pn), :], hb.at[i], sem
                        ).wait()

            return _Batch()

        def compute(slot, p):
            bufs = hbufs[slot]

            def row(r, _):
                for l in range(lanes):
                    sl = pl.ds(l * 16, 16)
                    a = wv[0] * bufs[0, r, sl]
                    for i in range(1, c):
                        a = a + wv[i] * bufs[i, r, sl]
                    accbuf[r, sl] = a
                return 0

            lax.fori_loop(0, pn, row, 0)
            pltpu.sync_copy(accbuf, out_ref.at[pl.ds(obase + p * pn, pn), :])

        copy(0, 0).start()

        def pair(p2, _):
            p = p2 * 2

            copy(1, p + 1).start()
            copy(0, p).wait()
            compute(0, p)

            @pl.when(p + 2 < pieces)
            def _():
                copy(0, p + 2).start()

            copy(1, p + 1).wait()
            compute(1, p + 1)
            return 0

        lax.fori_loop(0, pieces // 2, pair, 0)

    return sc_pass2


def kernel(h, W, b_lin, attn_vector, num_confs):
    del num_confs
    c, n, f = h.shape
    fo = W.shape[1]

    bn1 = _pick_block(n, 2048)
    nb1 = n // bn1
    wexp = pl.pallas_call(
        _pass1_kernel,
        grid=(nb1,),
        in_specs=[
            pl.BlockSpec((c, bn1, f), lambda i: (0, i, 0)),
            pl.BlockSpec((f, fo), lambda i: (0, 0)),
            pl.BlockSpec((1, fo), lambda i: (0, 0)),
            pl.BlockSpec((1, fo), lambda i: (0, 0)),
        ],
        out_specs=pl.BlockSpec((c, fo), lambda i: (0, 0)),
        out_shape=jax.ShapeDtypeStruct((c, fo), jnp.float32),
        scratch_shapes=[pltpu.VMEM((c, fo), jnp.float32),
                        pltpu.VMEM((c, fo), jnp.float32),
                        pltpu.VMEM((c, c * bn1), jnp.bfloat16)],
        compiler_params=pltpu.CompilerParams(
            dimension_semantics=("arbitrary",),
        ),
    )(h, W, b_lin.reshape(1, fo), attn_vector.reshape(1, fo))

    gran = _NW * _PN * 2
    n_sc = int(n * _SC_FRAC) // gran * gran
    n1 = n - n_sc

    out_sc = None
    if n_sc:
        npw = n_sc // _NW
        pieces = npw // _PN
        sc_fn = _make_sc_pass2(c, f, n1, npw, pieces, _PN)
        out_sc = sc_fn(h, wexp)

    if n1 == 0:
        return out_sc

    bn2 = _pick_block(n, 2048)
    nb2 = -(-n1 // bn2)
    out_tc = pl.pallas_call(
        _pass2_tc_kernel,
        grid=(nb2,),
        in_specs=[
            pl.BlockSpec((c, fo), lambda i: (0, 0)),
            pl.BlockSpec((c, bn2, f), lambda i: (0, i, 0)),
        ],
        out_specs=pl.BlockSpec((bn2, f), lambda i: (i, 0)),
        out_shape=jax.ShapeDtypeStruct((n, f), jnp.float32),
        compiler_params=pltpu.CompilerParams(
            dimension_semantics=("arbitrary",),
        ),
    )(wexp, h)

    if out_sc is None:
        return out_tc
    return lax.dynamic_update_slice(out_tc, out_sc, (n1, 0))


# speedup vs baseline: 1.0204x; 1.0204x over previous
import functools

import jax
import jax.numpy as jnp
from jax import lax
from jax.experimental import pallas as pl
from jax.experimental.pallas import tpu as pltpu
from jax.experimental.pallas import tpu_sc as plsc

_SC_FRAC = 0.45
_PN = 32
_NW = 32


def _pick_block(n, cap=2048):
    for b in range(cap - cap % 8, 0, -8):
        if n % b == 0:
            return b
    for b in range(min(n, cap), 0, -1):
        if n % b == 0:
            return b
    return n


def _pass1_kernel(h_ref, w_ref, bl_ref, a_ref, wexp_ref,
                  acc_ref, comp_ref, bd_ref):
    i = pl.program_id(0)
    c, bn, f = h_ref.shape

    @pl.when(i == 0)
    def _():
        acc_ref[...] = jnp.zeros_like(acc_ref)
        comp_ref[...] = jnp.zeros_like(comp_ref)
        row = lax.broadcasted_iota(jnp.int32, (c, c * bn), 0)
        col = lax.broadcasted_iota(jnp.int32, (c, c * bn), 1)
        bd_ref[...] = (col // bn == row).astype(jnp.bfloat16)

    x = h_ref[...].reshape(c * bn, f)
    s = jnp.tanh(
        jnp.dot(x, w_ref[...], preferred_element_type=jnp.float32) + bl_ref[...]
    )
    colsum = jnp.dot(bd_ref[...], s.astype(jnp.bfloat16),
                     preferred_element_type=jnp.float32)
    a_r = a_ref[...].astype(jnp.bfloat16).astype(jnp.float32)
    t = jnp.sum(colsum * a_r, axis=1)
    tb = jnp.broadcast_to(t[:, None], acc_ref.shape)
    y = tb - comp_ref[...]
    tot = acc_ref[...] + y
    comp_ref[...] = (tot - acc_ref[...]) - y
    acc_ref[...] = tot

    @pl.when(i == pl.num_programs(0) - 1)
    def _():
        b = acc_ref[...] * (1.0 / c)
        m = jnp.max(b, axis=0, keepdims=True)
        e = jnp.exp(b - m)
        wexp_ref[...] = e / jnp.sum(e, axis=0, keepdims=True)


def _pass2_tc_kernel(wexp_ref, h_ref, out_ref):
    c = h_ref.shape[0]
    w = wexp_ref[...]
    hb = h_ref[...]
    acc = hb[0] * w[0]
    for i in range(1, c):
        acc = acc + hb[i] * w[i]
    out_ref[...] = acc


def _make_sc_pass2(c, f, node0, npw, pieces, pn):
    mesh = plsc.VectorSubcoreMesh(core_axis_name="c", subcore_axis_name="s")
    nc = 2
    lanes = f // 16

    @functools.partial(
        pl.kernel,
        out_type=jax.ShapeDtypeStruct((npw * _NW, f), jnp.float32),
        mesh=mesh,
        scratch_types=[
            pltpu.VMEM((c, f), jnp.float32),
            [pltpu.VMEM((c, pn, f), jnp.float32) for _ in range(2)],
            pltpu.VMEM((pn, f), jnp.float32),
            [pltpu.SemaphoreType.DMA for _ in range(2)],
        ],
    )
    def sc_pass2(h_ref, wexp_ref, out_ref, wbuf, hbufs, accbuf, sems):
        wid = lax.axis_index("s") * nc + lax.axis_index("c")
        base = node0 + wid * npw
        obase = wid * npw
        pltpu.sync_copy(wexp_ref, wbuf)
        wv = [wbuf[i, 0:16] for i in range(c)]

        def copy(slot, p):
            src = base + p * pn
            hb, sem = hbufs[slot], sems[slot]

            class _Batch:
                def start(self):
                    for i in range(c):
                        pltpu.make_async_copy(
                            h_ref.at[i, pl.ds(src, pn), :], hb.at[i], sem
                        ).start()

                def wait(self):
                    for i in range(c):
                        pltpu.make_async_copy(
                            h_ref.at[i, pl.ds(src, pn), :], hb.at[i], sem
                        ).wait()

            return _Batch()

        def compute(slot, p):
            bufs = hbufs[slot]

            def row(r, _):
                for l in range(lanes):
                    sl = pl.ds(l * 16, 16)
                    a = wv[0] * bufs[0, r, sl]
                    for i in range(1, c):
                        a = a + wv[i] * bufs[i, r, sl]
                    accbuf[r, sl] = a
                return 0

            lax.fori_loop(0, pn, row, 0)
            pltpu.sync_copy(accbuf, out_ref.at[pl.ds(obase + p * pn, pn), :])

        copy(0, 0).start()

        def pair(p2, _):
            p = p2 * 2

            copy(1, p + 1).start()
            copy(0, p).wait()
            compute(0, p)

            @pl.when(p + 2 < pieces)
            def _():
                copy(0, p + 2).start()

            copy(1, p + 1).wait()
            compute(1, p + 1)
            return 0

        lax.fori_loop(0, pieces // 2, pair, 0)

    return sc_pass2


def kernel(h, W, b_lin, attn_vector, num_confs):
    del num_confs
    c, n, f = h.shape
    fo = W.shape[1]

    bn1 = _pick_block(n, 2048)
    nb1 = n // bn1
    wexp = pl.pallas_call(
        _pass1_kernel,
        grid=(nb1,),
        in_specs=[
            pl.BlockSpec((c, bn1, f), lambda i: (0, i, 0)),
            pl.BlockSpec((f, fo), lambda i: (0, 0)),
            pl.BlockSpec((1, fo), lambda i: (0, 0)),
            pl.BlockSpec((1, fo), lambda i: (0, 0)),
        ],
        out_specs=pl.BlockSpec((c, fo), lambda i: (0, 0)),
        out_shape=jax.ShapeDtypeStruct((c, fo), jnp.float32),
        scratch_shapes=[pltpu.VMEM((c, fo), jnp.float32),
                        pltpu.VMEM((c, fo), jnp.float32),
                        pltpu.VMEM((c, c * bn1), jnp.bfloat16)],
        compiler_params=pltpu.CompilerParams(
            dimension_semantics=("arbitrary",),
        ),
    )(h, W, b_lin.reshape(1, fo), attn_vector.reshape(1, fo))

    gran = _NW * _PN * 2
    n_sc = int(n * _SC_FRAC) // gran * gran
    n1 = n - n_sc

    out_sc = None
    if n_sc:
        npw = n_sc // _NW
        pieces = npw // _PN
        sc_fn = _make_sc_pass2(c, f, n1, npw, pieces, _PN)
        out_sc = sc_fn(h, wexp)

    if n1 == 0:
        return out_sc

    bn2 = _pick_block(n, 2048)
    nb2 = -(-n1 // bn2)
    out_tc = pl.pallas_call(
        _pass2_tc_kernel,
        grid=(nb2,),
        in_specs=[
            pl.BlockSpec((c, fo), lambda i: (0, 0)),
            pl.BlockSpec((c, bn2, f), lambda i: (0, i, 0)),
        ],
        out_specs=pl.BlockSpec((bn2, f), lambda i: (i, 0)),
        out_shape=jax.ShapeDtypeStruct((n, f), jnp.float32),
        compiler_params=pltpu.CompilerParams(
            dimension_semantics=("arbitrary",),
        ),
    )(wexp, h)

    if out_sc is None:
        return out_tc
    return lax.dynamic_update_slice(out_tc, out_sc, (n1, 0))
